# Initial kernel scaffold; baseline (speedup 1.0000x reference)
import jax
import jax.numpy as jnp
from jax.experimental import pallas as pl


def kernel(x, edge_attr, edge_index, W1, b1, root1, bias1, W2, b2, root2, bias2, W3, b3, root3, bias3):
    raise NotImplementedError("write your pallas kernel here")


# speedup vs baseline: 3.6089x; 3.6089x over previous
import jax
import jax.numpy as jnp
from jax import lax
from jax.experimental import pallas as pl

_N = 35
_E = 1190
_EP = 1280
_V = 6
_IB = 8


def _dot(a, b):
    return lax.dot_general(a, b, (((1,), (0,)), ((), ())),
                           preferred_element_type=jnp.float32)


def _nnconv_big(ea, xj, W_ref, b_ref, kd):
    bw = _IB * 128

    def body(i0, acc):
        wv = W_ref[:, pl.ds(i0 * bw, bw)]
        bv = b_ref[:, pl.ds(i0 * bw, bw)]
        genb = jnp.maximum(_dot(ea, wv) + bv, 0.0)
        sel = (kd == i0 * _IB).astype(jnp.float32)
        xb = _dot(xj, sel)
        p = xb * genb
        for di in range(_IB):
            acc = acc + p[:, di * 128:(di + 1) * 128]
        return acc

    return lax.fori_loop(0, 128 // _IB, body,
                         jnp.zeros((_EP, 128), jnp.float32))


def _dgn_kernel(ea_ref, src_ref, dst_ref, x_ref,
                W1_ref, b1_ref, root1_ref, bias1_ref,
                W2_ref, b2_ref, root2_ref, bias2_ref,
                W3_ref, b3_ref, root3_ref, bias3_ref,
                out_ref):
    f32 = jnp.float32
    ea = ea_ref[:]
    src = src_ref[:]
    dst = dst_ref[:]

    col = lax.broadcasted_iota(jnp.int32, (_EP, _N), 1)
    G = (src == col).astype(f32)
    row = lax.broadcasted_iota(jnp.int32, (_N, _EP), 0)
    S = (row == dst).astype(f32)
    cnt = jnp.sum(S, axis=1, keepdims=True)
    inv = 1.0 / jnp.maximum(cnt, 1.0)

    bw = _IB * 128
    k_i = lax.broadcasted_iota(jnp.int32, (128, bw), 0)
    c_i = lax.broadcasted_iota(jnp.int32, (128, bw), 1)
    kd = k_i - c_i // 128

    x0 = x_ref[:]
    gen = jnp.maximum(_dot(ea, W1_ref[:]) + b1_ref[:], 0.0)
    xj = _dot(G, x0)
    msg = xj * gen
    agg = _dot(S, msg) * inv
    h = jnp.maximum(_dot(x0, root1_ref[:]) + agg + bias1_ref[:], 0.0)

    xj = _dot(G, h)
    msg = _nnconv_big(ea, xj, W2_ref, b2_ref, kd)
    agg = _dot(S, msg) * inv
    h = jnp.maximum(_dot(h, root2_ref[:]) + agg + bias2_ref[:], 0.0)

    xj = _dot(G, h)
    msg = _nnconv_big(ea, xj, W3_ref, b3_ref, kd)
    agg = _dot(S, msg) * inv
    h = jnp.maximum(_dot(h, root3_ref[:]) + agg + bias3_ref[:], 0.0)

    d = jnp.abs(h[:, None, :] - h[None, :, :])
    out_ref[:] = jnp.sum(d, axis=2)


@jax.jit
def kernel(x, edge_attr, edge_index, W1, b1, root1, bias1,
           W2, b2, root2, bias2, W3, b3, root3, bias3):
    f32 = jnp.float32
    ea = jnp.zeros((_EP, _V), f32).at[:_E].set(edge_attr)
    src = jnp.full((_EP, 1), _N, jnp.int32).at[:_E, 0].set(edge_index[0])
    dst = jnp.full((1, _EP), _N, jnp.int32).at[0, :_E].set(edge_index[1])

    W3p = jnp.pad(W3.reshape(_V, 128, 64), ((0, 0), (0, 0), (0, 64))).reshape(_V, 128 * 128)
    b3p = jnp.pad(b3.reshape(128, 64), ((0, 0), (0, 64))).reshape(1, 128 * 128)
    root3p = jnp.pad(root3, ((0, 0), (0, 64)))
    bias3p = jnp.pad(bias3, ((0, 64))).reshape(1, 128)

    out = pl.pallas_call(
        _dgn_kernel,
        out_shape=jax.ShapeDtypeStruct((_N, _N), f32),
    )(ea, src, dst, x,
      W1, b1.reshape(1, -1), root1, bias1.reshape(1, -1),
      W2, b2.reshape(1, -1), root2, bias2.reshape(1, -1),
      W3p, b3p, root3p, bias3p)
    return out


# speedup vs baseline: 3.6196x; 1.0030x over previous
import jax
import jax.numpy as jnp
from jax import lax
from jax.experimental import pallas as pl

_N = 35
_E = 1190
_EP = 1280
_V = 6
_VA = 7
_IB = 16


def _dot(a, b):
    return lax.dot_general(a, b, (((1,), (0,)), ((), ())),
                           preferred_element_type=jnp.float32)


def _nnconv_big(ea, xj, W_ref, kd):
    bw = _IB * 128

    def body(i0, acc):
        wv = W_ref[:, pl.ds(i0 * bw, bw)]
        genb = jnp.maximum(_dot(ea, wv), 0.0)
        sel = (kd == i0 * _IB).astype(jnp.float32)
        xb = _dot(xj, sel)
        p = xb * genb
        for di in range(_IB):
            acc = acc + p[:, di * 128:(di + 1) * 128]
        return acc

    return lax.fori_loop(0, 128 // _IB, body,
                         jnp.zeros((_EP, 128), jnp.float32))


def _dgn_kernel(ea_ref, src_ref, dst_ref, x_ref,
                W1_ref, root1_ref, bias1_ref,
                W2_ref, root2_ref, bias2_ref,
                W3_ref, root3_ref, bias3_ref,
                out_ref):
    f32 = jnp.float32
    ea = ea_ref[:]
    src = src_ref[:]
    dst = dst_ref[:]

    col = lax.broadcasted_iota(jnp.int32, (_EP, _N), 1)
    G = (src == col).astype(f32)
    row = lax.broadcasted_iota(jnp.int32, (_N, _EP), 0)
    S = (row == dst).astype(f32)
    cnt = jnp.sum(S, axis=1, keepdims=True)
    inv = 1.0 / jnp.maximum(cnt, 1.0)

    bw = _IB * 128
    k_i = lax.broadcasted_iota(jnp.int32, (128, bw), 0)
    c_i = lax.broadcasted_iota(jnp.int32, (128, bw), 1)
    kd = k_i - c_i // 128

    x0 = x_ref[:]
    gen = jnp.maximum(_dot(ea, W1_ref[:]), 0.0)
    xj = _dot(G, x0)
    msg = xj * gen
    agg = _dot(S, msg) * inv
    h = jnp.maximum(_dot(x0, root1_ref[:]) + agg + bias1_ref[:], 0.0)

    xj = _dot(G, h)
    msg = _nnconv_big(ea, xj, W2_ref, kd)
    agg = _dot(S, msg) * inv
    h = jnp.maximum(_dot(h, root2_ref[:]) + agg + bias2_ref[:], 0.0)

    xj = _dot(G, h)
    msg = _nnconv_big(ea, xj, W3_ref, kd)
    agg = _dot(S, msg) * inv
    h = jnp.maximum(_dot(h, root3_ref[:]) + agg + bias3_ref[:], 0.0)

    d = jnp.abs(h[:, None, :] - h[None, :, :])
    out_ref[:] = jnp.sum(d, axis=2)


@jax.jit
def kernel(x, edge_attr, edge_index, W1, b1, root1, bias1,
           W2, b2, root2, bias2, W3, b3, root3, bias3):
    f32 = jnp.float32
    ea = jnp.zeros((_EP, _VA), f32).at[:_E, :_V].set(edge_attr).at[:, _V].set(1.0)
    src = jnp.full((_EP, 1), _N, jnp.int32).at[:_E, 0].set(edge_index[0])
    dst = jnp.full((1, _EP), _N, jnp.int32).at[0, :_E].set(edge_index[1])

    W1a = jnp.concatenate([W1, b1.reshape(1, -1)], axis=0)
    W2a = jnp.concatenate([W2, b2.reshape(1, -1)], axis=0)
    W3b = jnp.concatenate([W3, b3.reshape(1, -1)], axis=0)
    W3a = jnp.pad(W3b.reshape(_VA, 128, 64), ((0, 0), (0, 0), (0, 64))).reshape(_VA, 128 * 128)
    root3p = jnp.pad(root3, ((0, 0), (0, 64)))
    bias3p = jnp.pad(bias3, ((0, 64))).reshape(1, 128)

    out = pl.pallas_call(
        _dgn_kernel,
        out_shape=jax.ShapeDtypeStruct((_N, _N), f32),
    )(ea, src, dst, x,
      W1a, root1, bias1.reshape(1, -1),
      W2a, root2, bias2.reshape(1, -1),
      W3a, root3p, bias3p)
    return out
